# baseline (speedup 1.0000x reference)
import functools

import jax
import jax.numpy as jnp
from jax import lax
from jax.experimental import pallas as pl
from jax.experimental.pallas import tpu as pltpu
from jax.experimental.pallas import tpu_sc as plsc

EPS = 1e-15
NC = 2
NS = 16
CA = 40
GB = 25
HW = 16


def _deg_hist_kernel(NP, E):
    ept = E // (NC * NS)
    rpt = NP // NS
    nch = ept // CA
    nblk = nch // GB

    def body(col2_hbm, ones_hbm, zeros_hbm, degp_hbm,
             acc, cidx2, ones_v, sem):
        c = lax.axis_index("c")
        s = lax.axis_index("s")
        pltpu.sync_copy(zeros_hbm.at[pl.ds(s * rpt, rpt)], acc.at[pl.ds(s * rpt, rpt)])
        pltpu.sync_copy(ones_hbm, ones_v)
        plsc.subcore_barrier()
        cb0 = (c * NS + s) * nch

        def block(b, carry):
            pltpu.sync_copy(col2_hbm.at[pl.ds(cb0 + b * GB, GB), :], cidx2)

            def chunk(g, icarry):
                pltpu.async_copy(ones_v, acc.at[cidx2.at[g]], sem, add=True)
                return icarry

            lax.fori_loop(0, GB, chunk, 0)

            def drain(g, icarry):
                pltpu.make_async_copy(ones_v, acc.at[cidx2.at[g]], sem).wait()
                return icarry

            lax.fori_loop(0, GB, drain, 0)
            return carry

        lax.fori_loop(0, nblk, block, 0)
        plsc.subcore_barrier()
        pltpu.sync_copy(acc.at[pl.ds(s * rpt, rpt)],
                        degp_hbm.at[c, pl.ds(s * rpt, rpt)])

    mesh = plsc.VectorSubcoreMesh(core_axis_name="c", subcore_axis_name="s")
    return pl.kernel(
        body,
        out_type=jax.ShapeDtypeStruct((NC, NP, HW), jnp.float32),
        mesh=mesh,
        compiler_params=pltpu.CompilerParams(use_tc_tiling_on_sc=False),
        scratch_types=[
            pltpu.VMEM_SHARED((NP, HW), jnp.float32),
            pltpu.VMEM((GB, CA), jnp.int32),
            pltpu.VMEM((CA, HW), jnp.float32),
            pltpu.SemaphoreType.DMA,
        ],
    )


def _edge_kernel(NP, E, OUT, KG):
    ept = E // (NC * NS)
    rpt = NP // NS
    nj = OUT // 16
    nch = ept // CA
    nblk = nch // GB
    hf = CA // 2

    def body(row2_hbm, col3_hbm, gp_hbm, hh_hbm, zeros_hbm, outp_hbm,
             acc, ridx2, cidx3, rows0, rows1, hh0, hh1, msg0, msg1,
             sg0, sg1, sh0, sh1, sm0, sm1):
        c = lax.axis_index("c")
        s = lax.axis_index("s")
        pltpu.sync_copy(zeros_hbm.at[pl.ds(s * rpt, rpt)], acc.at[pl.ds(s * rpt, rpt)])
        plsc.subcore_barrier()
        cb0 = (c * NS + s) * nch
        rows = (rows0, rows1)
        hhs = (hh0, hh1)
        msgs = (msg0, msg1)
        sgs = (sg0, sg1)
        shs = (sh0, sh1)
        sms = (sm0, sm1)

        def wait_scatter(half):
            pltpu.make_async_copy(msgs[half], acc.at[cidx3.at[0, 0]],
                                  sms[half]).wait()

        def issue(g, t):
            pltpu.async_copy(gp_hbm.at[ridx2.at[g]], rows[t], sgs[t])
            pltpu.async_copy(hh_hbm.at[cidx3.at[g, 0]],
                             hhs[t].at[pl.ds(0, hf)], shs[t])
            pltpu.async_copy(hh_hbm.at[cidx3.at[g, 1]],
                             hhs[t].at[pl.ds(hf, hf)], shs[t])

        def wait(t):
            pltpu.make_async_copy(gp_hbm.at[ridx2.at[0]], rows[t], sgs[t]).wait()
            pltpu.make_async_copy(hh_hbm.at[cidx3.at[0, 0]],
                                  hhs[t].at[pl.ds(0, hf)], shs[t]).wait()
            pltpu.make_async_copy(hh_hbm.at[cidx3.at[0, 0]],
                                  hhs[t].at[pl.ds(0, hf)], shs[t]).wait()

        def consume(g, t):
            rv = rows[t]
            hv_ref = hhs[t]

            for half in (0, 1):
                msg_v = msgs[half]

                @pl.when(g >= 1)
                def _():
                    wait_scatter(half)

                @plsc.parallel_loop(0, hf, unroll=4)
                def edge(e):
                    es = e + half * hf
                    hv = hv_ref[es, pl.ds(0, HW)]
                    w0 = hv[0]
                    w1 = hv[1]
                    w2 = hv[2]
                    w3 = hv[3]
                    for j in range(nj):
                        v = rv[es, pl.ds(j * 16, 16)] * w0
                        v = v + rv[es, pl.ds(OUT + j * 16, 16)] * w1
                        v = v + rv[es, pl.ds(2 * OUT + j * 16, 16)] * w2
                        v = v + rv[es, pl.ds(3 * OUT + j * 16, 16)] * w3
                        msg_v[e, pl.ds(j * 16, 16)] = v

                pltpu.async_copy(msg_v, acc.at[cidx3.at[g, half]],
                                 sms[half], add=True)

        def block(b, carry):
            pltpu.sync_copy(row2_hbm.at[pl.ds(cb0 + b * GB, GB), :], ridx2)
            pltpu.sync_copy(col3_hbm.at[pl.ds(cb0 + b * GB, GB), :, :], cidx3)
            issue(0, 0)

            def pair(p, icarry):
                for t in (0, 1):
                    g = 2 * p + t
                    wait(t)
                    issue(g + 1, 1 - t)
                    consume(g, t)
                return icarry

            lax.fori_loop(0, (GB - 1) // 2, pair, 0)
            wait((GB - 1) % 2)
            consume(GB - 1, (GB - 1) % 2)
            wait_scatter(0)
            wait_scatter(1)
            return carry

        lax.fori_loop(0, nblk, block, 0)
        plsc.subcore_barrier()
        pltpu.sync_copy(acc.at[pl.ds(s * rpt, rpt)],
                        outp_hbm.at[c, pl.ds(s * rpt, rpt)])

    mesh = plsc.VectorSubcoreMesh(core_axis_name="c", subcore_axis_name="s")
    return pl.kernel(
        body,
        out_type=jax.ShapeDtypeStruct((NC, NP, OUT), jnp.float32),
        mesh=mesh,
        compiler_params=pltpu.CompilerParams(use_tc_tiling_on_sc=False),
        scratch_types=[
            pltpu.VMEM_SHARED((NP, OUT), jnp.float32),
            pltpu.VMEM((GB, CA), jnp.int32),
            pltpu.VMEM((GB, 2, CA // 2), jnp.int32),
            pltpu.VMEM((CA, KG * OUT), jnp.float32),
            pltpu.VMEM((CA, KG * OUT), jnp.float32),
            pltpu.VMEM((CA, HW), jnp.float32),
            pltpu.VMEM((CA, HW), jnp.float32),
            pltpu.VMEM((CA // 2, OUT), jnp.float32),
            pltpu.VMEM((CA // 2, OUT), jnp.float32),
            pltpu.SemaphoreType.DMA,
            pltpu.SemaphoreType.DMA,
            pltpu.SemaphoreType.DMA,
            pltpu.SemaphoreType.DMA,
            pltpu.SemaphoreType.DMA,
            pltpu.SemaphoreType.DMA,
        ],
    )


def _tables_body(x_ref, w_ref, root_ref, bias_ref, degp_ref, coef_ref,
                 gp_ref, hh_ref, xrb_ref, KG, OUT):
    x = x_ref[...]
    g = jnp.dot(x, w_ref[...], preferred_element_type=jnp.float32)
    xrb_ref[...] = (jnp.dot(x, root_ref[...], preferred_element_type=jnp.float32)
                    + bias_ref[...])
    deg = degp_ref[0, :, 0:1] + degp_ref[1, :, 0:1]
    dis = jax.lax.rsqrt(jnp.maximum(deg, 1.0))
    coef = coef_ref[...]
    f = jnp.exp(coef[1:2, :] * (dis - coef[0:1, :]) ** 2)
    h = jnp.exp(coef[3:4, :] * (dis - coef[2:3, :]) ** 2)
    for k in range(KG):
        gp_ref[:, k * OUT:(k + 1) * OUT] = g[:, k * OUT:(k + 1) * OUT] * f[:, k:k + 1]
    hh_ref[...] = jnp.concatenate([h, h, h, h], axis=1)


def _finish_body(x_ref, p_ref, degp_ref, xrb_ref, o_ref):
    deg = degp_ref[0, :, 0:1] + degp_ref[1, :, 0:1]
    agg = (p_ref[0] + p_ref[1]) / jnp.maximum(deg, 1.0)
    conv = agg + xrb_ref[...]
    o_ref[...] = x_ref[...] + jnp.maximum(conv, 0.0)


def kernel(x, edge_index, W, mu, sigma, root, bias):
    N, IN = x.shape
    E = edge_index.shape[1]
    OUT = root.shape[1]
    KG = W.shape[1] // OUT

    NP = ((N + 127) // 128) * 128
    row2 = edge_index[0].reshape(E // CA, CA)
    col2 = edge_index[1].reshape(E // CA, CA)
    col3 = edge_index[1].reshape(E // CA, 2, CA // 2)
    ones8 = jnp.ones((CA, HW), jnp.float32)
    zeros8 = jnp.zeros((NP, HW), jnp.float32)
    zerosO = jnp.zeros((NP, OUT), jnp.float32)
    inv = -0.5 / (sigma * sigma + EPS)
    coef = jnp.stack([mu[:, 0], inv[:, 0], mu[:, 1], inv[:, 1]], axis=0)

    degp = _deg_hist_kernel(NP, E)(col2, ones8, zeros8)

    B = 400
    grid = (N // B,)
    gp, hh, xrb = pl.pallas_call(
        functools.partial(_tables_body, KG=KG, OUT=OUT),
        grid=grid,
        in_specs=[
            pl.BlockSpec((B, IN), lambda i: (i, 0)),
            pl.BlockSpec((IN, KG * OUT), lambda i: (0, 0)),
            pl.BlockSpec((IN, OUT), lambda i: (0, 0)),
            pl.BlockSpec((1, OUT), lambda i: (0, 0)),
            pl.BlockSpec((NC, B, HW), lambda i: (0, i, 0)),
            pl.BlockSpec((4, KG), lambda i: (0, 0)),
        ],
        out_specs=[
            pl.BlockSpec((B, KG * OUT), lambda i: (i, 0)),
            pl.BlockSpec((B, HW), lambda i: (i, 0)),
            pl.BlockSpec((B, OUT), lambda i: (i, 0)),
        ],
        out_shape=[
            jax.ShapeDtypeStruct((N, KG * OUT), jnp.float32),
            jax.ShapeDtypeStruct((N, HW), jnp.float32),
            jax.ShapeDtypeStruct((N, OUT), jnp.float32),
        ],
    )(x, W, root, bias.reshape(1, OUT), degp, coef)

    outp = _edge_kernel(NP, E, OUT, KG)(row2, col3, gp, hh, zerosO)

    out = pl.pallas_call(
        _finish_body,
        grid=grid,
        in_specs=[
            pl.BlockSpec((B, IN), lambda i: (i, 0)),
            pl.BlockSpec((NC, B, OUT), lambda i: (0, i, 0)),
            pl.BlockSpec((NC, B, HW), lambda i: (0, i, 0)),
            pl.BlockSpec((B, OUT), lambda i: (i, 0)),
        ],
        out_specs=pl.BlockSpec((B, OUT), lambda i: (i, 0)),
        out_shape=jax.ShapeDtypeStruct((N, OUT), jnp.float32),
    )(x, outp, degp, xrb)
    return out


# speedup vs baseline: 1.1314x; 1.1314x over previous
import functools

import jax
import jax.numpy as jnp
from jax import lax
from jax.experimental import pallas as pl
from jax.experimental.pallas import tpu as pltpu
from jax.experimental.pallas import tpu_sc as plsc

EPS = 1e-15
NC = 2
NS = 16
CA = 40
GB = 25
HW = 16


def _deg_hist_kernel(NP, E):
    ept = E // (NC * NS)
    rpt = NP // NS
    nch = ept // CA
    nblk = nch // GB

    def body(col2_hbm, ones_hbm, zeros_hbm, degp_hbm,
             acc, cidx2, ones_v, sem):
        c = lax.axis_index("c")
        s = lax.axis_index("s")
        pltpu.sync_copy(zeros_hbm.at[pl.ds(s * rpt, rpt)], acc.at[pl.ds(s * rpt, rpt)])
        pltpu.sync_copy(ones_hbm, ones_v)
        plsc.subcore_barrier()
        cb0 = (c * NS + s) * nch

        def block(b, carry):
            pltpu.sync_copy(col2_hbm.at[pl.ds(cb0 + b * GB, GB), :], cidx2)

            def chunk(g, icarry):
                pltpu.async_copy(ones_v, acc.at[cidx2.at[g]], sem, add=True)
                return icarry

            lax.fori_loop(0, GB, chunk, 0)

            def drain(g, icarry):
                pltpu.make_async_copy(ones_v, acc.at[cidx2.at[g]], sem).wait()
                return icarry

            lax.fori_loop(0, GB, drain, 0)
            return carry

        lax.fori_loop(0, nblk, block, 0)
        plsc.subcore_barrier()
        pltpu.sync_copy(acc.at[pl.ds(s * rpt, rpt)],
                        degp_hbm.at[c, pl.ds(s * rpt, rpt)])

    mesh = plsc.VectorSubcoreMesh(core_axis_name="c", subcore_axis_name="s")
    return pl.kernel(
        body,
        out_type=jax.ShapeDtypeStruct((NC, NP, HW), jnp.float32),
        mesh=mesh,
        compiler_params=pltpu.CompilerParams(use_tc_tiling_on_sc=False),
        scratch_types=[
            pltpu.VMEM_SHARED((NP, HW), jnp.float32),
            pltpu.VMEM((GB, CA), jnp.int32),
            pltpu.VMEM((CA, HW), jnp.float32),
            pltpu.SemaphoreType.DMA,
        ],
    )


def _edge_kernel(NP, E, OUT, KG):
    ept = E // (NC * NS)
    rpt = NP // NS
    nj = OUT // 16
    nch = ept // CA
    nblk = nch // GB

    def body(row2_hbm, col2_hbm, gp_hbm, hh_hbm, zeros_hbm, outp_hbm,
             acc, ridx2, cidx2, rows0, rows1, hh0, hh1, msg_v,
             sg0, sg1, sh0, sh1):
        c = lax.axis_index("c")
        s = lax.axis_index("s")
        pltpu.sync_copy(zeros_hbm.at[pl.ds(s * rpt, rpt)], acc.at[pl.ds(s * rpt, rpt)])
        plsc.subcore_barrier()
        cb0 = (c * NS + s) * nch
        rows = (rows0, rows1)
        hhs = (hh0, hh1)
        sgs = (sg0, sg1)
        shs = (sh0, sh1)

        def issue(g, t):
            pltpu.async_copy(gp_hbm.at[ridx2.at[g]], rows[t], sgs[t])
            pltpu.async_copy(hh_hbm.at[cidx2.at[g]], hhs[t], shs[t])

        def wait(t):
            pltpu.make_async_copy(gp_hbm.at[ridx2.at[0]], rows[t], sgs[t]).wait()
            pltpu.make_async_copy(hh_hbm.at[cidx2.at[0]], hhs[t], shs[t]).wait()

        def consume(g, t):
            rv = rows[t]
            hv_ref = hhs[t]

            @plsc.parallel_loop(0, CA, unroll=8)
            def edge(e):
                hv = hv_ref[e, pl.ds(0, HW)]
                w0 = hv[0]
                w1 = hv[1]
                w2 = hv[2]
                w3 = hv[3]
                for j in range(nj):
                    v = rv[e, pl.ds(j * 16, 16)] * w0
                    v = v + rv[e, pl.ds(OUT + j * 16, 16)] * w1
                    v = v + rv[e, pl.ds(2 * OUT + j * 16, 16)] * w2
                    v = v + rv[e, pl.ds(3 * OUT + j * 16, 16)] * w3
                    msg_v[e, pl.ds(j * 16, 16)] = v
            pltpu.sync_copy(msg_v, acc.at[cidx2.at[g]], add=True)

        def block(b, carry):
            pltpu.sync_copy(row2_hbm.at[pl.ds(cb0 + b * GB, GB), :], ridx2)
            pltpu.sync_copy(col2_hbm.at[pl.ds(cb0 + b * GB, GB), :], cidx2)
            issue(0, 0)

            def pair(p, icarry):
                for t in (0, 1):
                    g = 2 * p + t
                    wait(t)
                    issue(g + 1, 1 - t)
                    consume(g, t)
                return icarry

            lax.fori_loop(0, (GB - 1) // 2, pair, 0)
            wait((GB - 1) % 2)
            consume(GB - 1, (GB - 1) % 2)
            return carry

        lax.fori_loop(0, nblk, block, 0)
        plsc.subcore_barrier()
        pltpu.sync_copy(acc.at[pl.ds(s * rpt, rpt)],
                        outp_hbm.at[c, pl.ds(s * rpt, rpt)])

    mesh = plsc.VectorSubcoreMesh(core_axis_name="c", subcore_axis_name="s")
    return pl.kernel(
        body,
        out_type=jax.ShapeDtypeStruct((NC, NP, OUT), jnp.float32),
        mesh=mesh,
        compiler_params=pltpu.CompilerParams(use_tc_tiling_on_sc=False),
        scratch_types=[
            pltpu.VMEM_SHARED((NP, OUT), jnp.float32),
            pltpu.VMEM((GB, CA), jnp.int32),
            pltpu.VMEM((GB, CA), jnp.int32),
            pltpu.VMEM((CA, KG * OUT), jnp.float32),
            pltpu.VMEM((CA, KG * OUT), jnp.float32),
            pltpu.VMEM((CA, HW), jnp.float32),
            pltpu.VMEM((CA, HW), jnp.float32),
            pltpu.VMEM((CA, OUT), jnp.float32),
            pltpu.SemaphoreType.DMA,
            pltpu.SemaphoreType.DMA,
            pltpu.SemaphoreType.DMA,
            pltpu.SemaphoreType.DMA,
        ],
    )


def _tables_body(x_ref, w_ref, root_ref, bias_ref, degp_ref, coef_ref,
                 gp_ref, hh_ref, xrb_ref, KG, OUT):
    x = x_ref[...]
    g = jnp.dot(x, w_ref[...], preferred_element_type=jnp.float32)
    xrb_ref[...] = (jnp.dot(x, root_ref[...], preferred_element_type=jnp.float32)
                    + bias_ref[...])
    deg = degp_ref[0, :, 0:1] + degp_ref[1, :, 0:1]
    dis = jax.lax.rsqrt(jnp.maximum(deg, 1.0))
    coef = coef_ref[...]
    f = jnp.exp(coef[1:2, :] * (dis - coef[0:1, :]) ** 2)
    h = jnp.exp(coef[3:4, :] * (dis - coef[2:3, :]) ** 2)
    for k in range(KG):
        gp_ref[:, k * OUT:(k + 1) * OUT] = g[:, k * OUT:(k + 1) * OUT] * f[:, k:k + 1]
    hh_ref[...] = jnp.concatenate([h, h, h, h], axis=1)


def _finish_body(x_ref, p_ref, degp_ref, xrb_ref, o_ref):
    deg = degp_ref[0, :, 0:1] + degp_ref[1, :, 0:1]
    agg = (p_ref[0] + p_ref[1]) / jnp.maximum(deg, 1.0)
    conv = agg + xrb_ref[...]
    o_ref[...] = x_ref[...] + jnp.maximum(conv, 0.0)


def kernel(x, edge_index, W, mu, sigma, root, bias):
    N, IN = x.shape
    E = edge_index.shape[1]
    OUT = root.shape[1]
    KG = W.shape[1] // OUT

    NP = ((N + 127) // 128) * 128
    row2 = edge_index[0].reshape(E // CA, CA)
    col2 = edge_index[1].reshape(E // CA, CA)
    ones8 = jnp.ones((CA, HW), jnp.float32)
    zeros8 = jnp.zeros((NP, HW), jnp.float32)
    zerosO = jnp.zeros((NP, OUT), jnp.float32)
    inv = -0.5 / (sigma * sigma + EPS)
    coef = jnp.stack([mu[:, 0], inv[:, 0], mu[:, 1], inv[:, 1]], axis=0)

    degp = _deg_hist_kernel(NP, E)(col2, ones8, zeros8)

    B = 400
    grid = (N // B,)
    gp, hh, xrb = pl.pallas_call(
        functools.partial(_tables_body, KG=KG, OUT=OUT),
        grid=grid,
        in_specs=[
            pl.BlockSpec((B, IN), lambda i: (i, 0)),
            pl.BlockSpec((IN, KG * OUT), lambda i: (0, 0)),
            pl.BlockSpec((IN, OUT), lambda i: (0, 0)),
            pl.BlockSpec((1, OUT), lambda i: (0, 0)),
            pl.BlockSpec((NC, B, HW), lambda i: (0, i, 0)),
            pl.BlockSpec((4, KG), lambda i: (0, 0)),
        ],
        out_specs=[
            pl.BlockSpec((B, KG * OUT), lambda i: (i, 0)),
            pl.BlockSpec((B, HW), lambda i: (i, 0)),
            pl.BlockSpec((B, OUT), lambda i: (i, 0)),
        ],
        out_shape=[
            jax.ShapeDtypeStruct((N, KG * OUT), jnp.float32),
            jax.ShapeDtypeStruct((N, HW), jnp.float32),
            jax.ShapeDtypeStruct((N, OUT), jnp.float32),
        ],
    )(x, W, root, bias.reshape(1, OUT), degp, coef)

    outp = _edge_kernel(NP, E, OUT, KG)(row2, col2, gp, hh, zerosO)

    out = pl.pallas_call(
        _finish_body,
        grid=grid,
        in_specs=[
            pl.BlockSpec((B, IN), lambda i: (i, 0)),
            pl.BlockSpec((NC, B, OUT), lambda i: (0, i, 0)),
            pl.BlockSpec((NC, B, HW), lambda i: (0, i, 0)),
            pl.BlockSpec((B, OUT), lambda i: (i, 0)),
        ],
        out_specs=pl.BlockSpec((B, OUT), lambda i: (i, 0)),
        out_shape=jax.ShapeDtypeStruct((N, OUT), jnp.float32),
    )(x, outp, degp, xrb)
    return out


# speedup vs baseline: 1.1554x; 1.0212x over previous
import functools

import jax
import jax.numpy as jnp
from jax import lax
from jax.experimental import pallas as pl
from jax.experimental.pallas import tpu as pltpu
from jax.experimental.pallas import tpu_sc as plsc

EPS = 1e-15
NC = 2
NS = 16
CA = 40
GB = 25
HW = 16


def _deg_hist_kernel(NP, E):
    ept = E // (NC * NS)
    rpt = NP // NS
    nch = ept // CA
    nblk = nch // GB

    def body(ei3_hbm, ones_hbm, zeros_hbm, degp_hbm,
             acc, cidx2, ones_v, sem):
        c = lax.axis_index("c")
        s = lax.axis_index("s")
        pltpu.sync_copy(zeros_hbm.at[pl.ds(s * rpt, rpt)], acc.at[pl.ds(s * rpt, rpt)])
        pltpu.sync_copy(ones_hbm, ones_v)
        plsc.subcore_barrier()
        cb0 = (c * NS + s) * nch

        def block(b, carry):
            pltpu.sync_copy(ei3_hbm.at[1, pl.ds(cb0 + b * GB, GB), :], cidx2)

            def chunk(g, icarry):
                pltpu.async_copy(ones_v, acc.at[cidx2.at[g]], sem, add=True)
                return icarry

            lax.fori_loop(0, GB, chunk, 0)

            def drain(g, icarry):
                pltpu.make_async_copy(ones_v, acc.at[cidx2.at[g]], sem).wait()
                return icarry

            lax.fori_loop(0, GB, drain, 0)
            return carry

        lax.fori_loop(0, nblk, block, 0)
        plsc.subcore_barrier()
        pltpu.sync_copy(acc.at[pl.ds(s * rpt, rpt)],
                        degp_hbm.at[c, pl.ds(s * rpt, rpt)])

    mesh = plsc.VectorSubcoreMesh(core_axis_name="c", subcore_axis_name="s")
    return pl.kernel(
        body,
        out_type=jax.ShapeDtypeStruct((NC, NP, HW), jnp.float32),
        mesh=mesh,
        compiler_params=pltpu.CompilerParams(use_tc_tiling_on_sc=False),
        scratch_types=[
            pltpu.VMEM_SHARED((NP, HW), jnp.float32),
            pltpu.VMEM((GB, CA), jnp.int32),
            pltpu.VMEM((CA, HW), jnp.float32),
            pltpu.SemaphoreType.DMA,
        ],
    )


def _edge_kernel(NP, E, OUT, KG):
    ept = E // (NC * NS)
    rpt = NP // NS
    nj = OUT // 16
    nch = ept // CA
    nblk = nch // GB

    def body(ei3_hbm, gp_hbm, hh_hbm, zeros_hbm, outp_hbm,
             acc, ridx2, cidx2, rows0, rows1, hh0, hh1, msg_v,
             sg0, sg1, sh0, sh1):
        c = lax.axis_index("c")
        s = lax.axis_index("s")
        pltpu.sync_copy(zeros_hbm.at[pl.ds(s * rpt, rpt)], acc.at[pl.ds(s * rpt, rpt)])
        plsc.subcore_barrier()
        cb0 = (c * NS + s) * nch
        rows = (rows0, rows1)
        hhs = (hh0, hh1)
        sgs = (sg0, sg1)
        shs = (sh0, sh1)

        def issue(g, t):
            pltpu.async_copy(gp_hbm.at[ridx2.at[g]], rows[t], sgs[t])
            pltpu.async_copy(hh_hbm.at[cidx2.at[g]], hhs[t], shs[t])

        def wait(t):
            pltpu.make_async_copy(gp_hbm.at[ridx2.at[0]], rows[t], sgs[t]).wait()
            pltpu.make_async_copy(hh_hbm.at[cidx2.at[0]], hhs[t], shs[t]).wait()

        def consume(g, t):
            rv = rows[t]
            hv_ref = hhs[t]

            @plsc.parallel_loop(0, CA, unroll=8)
            def edge(e):
                hv = hv_ref[e, pl.ds(0, HW)]
                w0 = hv[0]
                w1 = hv[1]
                w2 = hv[2]
                w3 = hv[3]
                for j in range(nj):
                    v = rv[e, pl.ds(j * 16, 16)] * w0
                    v = v + rv[e, pl.ds(OUT + j * 16, 16)] * w1
                    v = v + rv[e, pl.ds(2 * OUT + j * 16, 16)] * w2
                    v = v + rv[e, pl.ds(3 * OUT + j * 16, 16)] * w3
                    msg_v[e, pl.ds(j * 16, 16)] = v
            pltpu.sync_copy(msg_v, acc.at[cidx2.at[g]], add=True)

        def block(b, carry):
            pltpu.sync_copy(ei3_hbm.at[0, pl.ds(cb0 + b * GB, GB), :], ridx2)
            pltpu.sync_copy(ei3_hbm.at[1, pl.ds(cb0 + b * GB, GB), :], cidx2)
            issue(0, 0)

            def pair(p, icarry):
                for t in (0, 1):
                    g = 2 * p + t
                    wait(t)
                    issue(g + 1, 1 - t)
                    consume(g, t)
                return icarry

            lax.fori_loop(0, (GB - 1) // 2, pair, 0)
            wait((GB - 1) % 2)
            consume(GB - 1, (GB - 1) % 2)
            return carry

        lax.fori_loop(0, nblk, block, 0)
        plsc.subcore_barrier()
        pltpu.sync_copy(acc.at[pl.ds(s * rpt, rpt)],
                        outp_hbm.at[c, pl.ds(s * rpt, rpt)])

    mesh = plsc.VectorSubcoreMesh(core_axis_name="c", subcore_axis_name="s")
    return pl.kernel(
        body,
        out_type=jax.ShapeDtypeStruct((NC, NP, OUT), jnp.float32),
        mesh=mesh,
        compiler_params=pltpu.CompilerParams(use_tc_tiling_on_sc=False),
        scratch_types=[
            pltpu.VMEM_SHARED((NP, OUT), jnp.float32),
            pltpu.VMEM((GB, CA), jnp.int32),
            pltpu.VMEM((GB, CA), jnp.int32),
            pltpu.VMEM((CA, KG * OUT), jnp.float32),
            pltpu.VMEM((CA, KG * OUT), jnp.float32),
            pltpu.VMEM((CA, HW), jnp.float32),
            pltpu.VMEM((CA, HW), jnp.float32),
            pltpu.VMEM((CA, OUT), jnp.float32),
            pltpu.SemaphoreType.DMA,
            pltpu.SemaphoreType.DMA,
            pltpu.SemaphoreType.DMA,
            pltpu.SemaphoreType.DMA,
        ],
    )


def _tables_body(x_ref, w_ref, root_ref, bias_ref, degp_ref, coef_ref,
                 gp_ref, hh_ref, xrb_ref, KG, OUT):
    x = x_ref[...]
    g = jnp.dot(x, w_ref[...], preferred_element_type=jnp.float32)
    xrb_ref[...] = (jnp.dot(x, root_ref[...], preferred_element_type=jnp.float32)
                    + bias_ref[...])
    deg = degp_ref[0, :, 0:1] + degp_ref[1, :, 0:1]
    dis = jax.lax.rsqrt(jnp.maximum(deg, 1.0))
    coef = coef_ref[...]
    f = jnp.exp(coef[1:2, :] * (dis - coef[0:1, :]) ** 2)
    h = jnp.exp(coef[3:4, :] * (dis - coef[2:3, :]) ** 2)
    for k in range(KG):
        gp_ref[:, k * OUT:(k + 1) * OUT] = g[:, k * OUT:(k + 1) * OUT] * f[:, k:k + 1]
    hh_ref[...] = jnp.concatenate([h, h, h, h], axis=1)


def _finish_body(x_ref, p_ref, degp_ref, xrb_ref, o_ref):
    deg = degp_ref[0, :, 0:1] + degp_ref[1, :, 0:1]
    agg = (p_ref[0] + p_ref[1]) / jnp.maximum(deg, 1.0)
    conv = agg + xrb_ref[...]
    o_ref[...] = x_ref[...] + jnp.maximum(conv, 0.0)


def kernel(x, edge_index, W, mu, sigma, root, bias):
    N, IN = x.shape
    E = edge_index.shape[1]
    OUT = root.shape[1]
    KG = W.shape[1] // OUT

    NP = ((N + 127) // 128) * 128
    ei3 = edge_index.reshape(2, E // CA, CA)
    ones8 = jnp.ones((CA, HW), jnp.float32)
    zeros8 = jnp.zeros((NP, HW), jnp.float32)
    zerosO = jnp.zeros((NP, OUT), jnp.float32)
    inv = -0.5 / (sigma * sigma + EPS)
    coef = jnp.stack([mu[:, 0], inv[:, 0], mu[:, 1], inv[:, 1]], axis=0)

    degp = _deg_hist_kernel(NP, E)(ei3, ones8, zeros8)

    B = 400
    grid = (N // B,)
    gp, hh, xrb = pl.pallas_call(
        functools.partial(_tables_body, KG=KG, OUT=OUT),
        grid=grid,
        in_specs=[
            pl.BlockSpec((B, IN), lambda i: (i, 0)),
            pl.BlockSpec((IN, KG * OUT), lambda i: (0, 0)),
            pl.BlockSpec((IN, OUT), lambda i: (0, 0)),
            pl.BlockSpec((1, OUT), lambda i: (0, 0)),
            pl.BlockSpec((NC, B, HW), lambda i: (0, i, 0)),
            pl.BlockSpec((4, KG), lambda i: (0, 0)),
        ],
        out_specs=[
            pl.BlockSpec((B, KG * OUT), lambda i: (i, 0)),
            pl.BlockSpec((B, HW), lambda i: (i, 0)),
            pl.BlockSpec((B, OUT), lambda i: (i, 0)),
        ],
        out_shape=[
            jax.ShapeDtypeStruct((N, KG * OUT), jnp.float32),
            jax.ShapeDtypeStruct((N, HW), jnp.float32),
            jax.ShapeDtypeStruct((N, OUT), jnp.float32),
        ],
    )(x, W, root, bias.reshape(1, OUT), degp, coef)

    outp = _edge_kernel(NP, E, OUT, KG)(ei3, gp, hh, zerosO)

    out = pl.pallas_call(
        _finish_body,
        grid=grid,
        in_specs=[
            pl.BlockSpec((B, IN), lambda i: (i, 0)),
            pl.BlockSpec((NC, B, OUT), lambda i: (0, i, 0)),
            pl.BlockSpec((NC, B, HW), lambda i: (0, i, 0)),
            pl.BlockSpec((B, OUT), lambda i: (i, 0)),
        ],
        out_specs=pl.BlockSpec((B, OUT), lambda i: (i, 0)),
        out_shape=jax.ShapeDtypeStruct((N, OUT), jnp.float32),
    )(x, outp, degp, xrb)
    return out
